# baseline (speedup 1.0000x reference)
import jax
import jax.numpy as jnp
from jax import lax
from jax.experimental import pallas as pl
from jax.experimental.pallas import tpu as pltpu
from jax.experimental.pallas import tpu_sc as plsc

H = 256
W = 256
ZB = 16
B = 2
C = 64
EMB = 32
M = 500000
HW = H * W
NSEG = B * HW
RANGE = 8192
NPASS = (NSEG // 2) // RANGE
DUMF = 512
NR = RANGE + DUMF
NZT = NR * ZB
MP = 524288
SHARE = MP // 16
CH = 512
NCH = SHARE // CH
TAIL = M - (M // CH) * CH
FSPAN = RANGE // 16
ZSPAN = RANGE * ZB // 16
SENT = 0x7FFFFFFF


def _sc_body(keys_hbm, feat_hbm, zf_hbm, sums_hbm, occz_hbm,
             kbuf, fbuf, loccb, rkeyb, fbix, zbix, dumf, dumz, onesb,
             dstg, accF, accZ, semk, semf, sems):
    c = lax.axis_index("c")
    s = lax.axis_index("s")
    share_base = s * SHARE
    lane = lax.iota(jnp.int32, 16)

    pltpu.sync_copy(zf_hbm, fbuf)
    zero16 = jnp.zeros((16,), jnp.float32)

    def zb_body(j, carry):
        dstg[pl.ds(j * 16, 16)] = zero16
        return carry

    lax.fori_loop(0, ZSPAN // 16, zb_body, 0)
    for j in range(CH // 16):
        onesb[pl.ds(j * 16, 16)] = jnp.full((16,), 1.0, jnp.float32)
    for q in range(FSPAN // 128):
        row = fbix.at[q]
        for j in range(8):
            row[pl.ds(j * 16, 16)] = lane + (s * FSPAN + q * 128 + j * 16)

    def zbix_body(q, carry):
        row = zbix.at[q]
        for j in range(8):
            row[pl.ds(j * 16, 16)] = lane + (s * ZSPAN + q * 128 + j * 16)
        return carry

    lax.fori_loop(0, ZSPAN // 128, zbix_body, 0)
    for q in range(DUMF // 128):
        dfrow = dumf.at[q]
        for j in range(8):
            e = lane + (q * 128 + j * 16)
            dfrow[pl.ds(j * 16, 16)] = RANGE + (e & (DUMF - 1))
    for q in range(4):
        row = dumz.at[q]
        for j in range(8):
            row[pl.ds(j * 16, 16)] = lane + (RANGE * ZB + q * 128 + j * 16)

    cps = []
    for q in range(FSPAN // 128):
        cps.append(pltpu.async_copy(fbuf.at[pl.ds(q * 128, 128), :],
                                    accF.at[fbix.at[q]], sems))
    for q in range(DUMF // 128):
        cps.append(pltpu.async_copy(fbuf.at[pl.ds(q * 128, 128), :],
                                    accF.at[dumf.at[q]], sems))
    for q in range(ZSPAN // 128):
        cps.append(pltpu.async_copy(dstg.at[pl.ds(q * 128, 128)],
                                    accZ.at[zbix.at[q]], sems))
    for q in range(4):
        cps.append(pltpu.async_copy(dstg.at[pl.ds(q * 128, 128)],
                                    accZ.at[dumz.at[q]], sems))
    for cp in cps:
        cp.wait()
    plsc.subcore_barrier()

    def do_pass(p, carry):
        rbase = c * (NSEG // 2) + p * RANGE
        kb16 = rbase * ZB

        def chunk_body(i, carry2):
            kbase = share_base + i * CH

            @pl.when(kbase < M)
            def _():
                ck = pltpu.async_copy(keys_hbm.at[pl.ds(kbase, CH)], kbuf,
                                      semk)
                full = (kbase + CH) <= M

                @pl.when(full)
                def _():
                    pltpu.async_copy(feat_hbm.at[pl.ds(kbase, CH), :], fbuf,
                                     semf)

                @pl.when(jnp.logical_not(full))
                def _():
                    pltpu.async_copy(feat_hbm.at[pl.ds(kbase, TAIL), :],
                                     fbuf.at[pl.ds(0, TAIL), :], semf)
                    pltpu.async_copy(feat_hbm.at[pl.ds(0, CH - TAIL), :],
                                     fbuf.at[pl.ds(TAIL, CH - TAIL), :],
                                     semf)

                ck.wait()
                for q in range(CH // 128):
                    lrow = loccb.at[q]
                    rrow = rkeyb.at[q]
                    for j in range(8):
                        off = q * 128 + j * 16
                        k = kbuf[pl.ds(off, 16)]
                        code = lax.shift_right_logical(k, 4)
                        m = jnp.logical_and(code >= rbase,
                                            code < rbase + RANGE)
                        l = lane + off
                        lrow[pl.ds(j * 16, 16)] = jnp.where(
                            m, code - rbase, RANGE + (l & (DUMF - 1)))
                        rrow[pl.ds(j * 16, 16)] = jnp.where(
                            m, k - kb16, RANGE * ZB + (l & 511))
                pltpu.make_async_copy(feat_hbm.at[pl.ds(0, CH), :], fbuf,
                                      semf).wait()
                adds = []
                for q in range(CH // 128):
                    adds.append(pltpu.async_copy(
                        fbuf.at[pl.ds(q * 128, 128), :],
                        accF.at[loccb.at[q]], sems, add=True))
                    adds.append(pltpu.async_copy(
                        onesb.at[pl.ds(q * 128, 128)],
                        accZ.at[rkeyb.at[q]], sems, add=True))
                for cp in adds:
                    cp.wait()

            return carry2

        lax.fori_loop(0, NCH, chunk_body, 0)
        plsc.subcore_barrier()
        gs = []
        for q in range(FSPAN // 128):
            gs.append(pltpu.async_copy(accF.at[fbix.at[q]],
                                       fbuf.at[pl.ds(q * 128, 128), :],
                                       sems))
        for q in range(ZSPAN // 128):
            gs.append(pltpu.async_copy(accZ.at[zbix.at[q]],
                                       dstg.at[pl.ds(q * 128, 128)], sems))
        for cp in gs:
            cp.wait()
        pltpu.sync_copy(fbuf,
                        sums_hbm.at[pl.ds(rbase + s * FSPAN, FSPAN), :])
        pltpu.sync_copy(dstg,
                        occz_hbm.at[pl.ds(kb16 + s * ZSPAN, ZSPAN)])
        plsc.subcore_barrier()
        return carry

    lax.fori_loop(0, NPASS, do_pass, 0)


_sc_scatter = pl.kernel(
    _sc_body,
    out_type=(jax.ShapeDtypeStruct((NSEG, C), jnp.float32),
              jax.ShapeDtypeStruct((NSEG * ZB,), jnp.float32)),
    mesh=plsc.VectorSubcoreMesh(core_axis_name="c", subcore_axis_name="s"),
    compiler_params=pltpu.CompilerParams(use_tc_tiling_on_sc=False),
    scratch_types=[
        pltpu.VMEM((CH,), jnp.int32),
        pltpu.VMEM((CH, C), jnp.float32),
        pltpu.VMEM((CH // 128, 128), jnp.int32),
        pltpu.VMEM((CH // 128, 128), jnp.int32),
        pltpu.VMEM((FSPAN // 128, 128), jnp.int32),
        pltpu.VMEM((ZSPAN // 128, 128), jnp.int32),
        pltpu.VMEM((DUMF // 128, 128), jnp.int32),
        pltpu.VMEM((4, 128), jnp.int32),
        pltpu.VMEM((CH,), jnp.float32),
        pltpu.VMEM((ZSPAN,), jnp.float32),
        pltpu.VMEM_SHARED((NR, C), jnp.float32),
        pltpu.VMEM_SHARED((NZT,), jnp.float32),
        pltpu.SemaphoreType.DMA,
        pltpu.SemaphoreType.DMA,
        pltpu.SemaphoreType.DMA,
    ],
)

GY = 8
GRID = B * H // GY
PB = RANGE // (GY * W)
HB = (NSEG // 2) // (GY * W)


def _tc_body(sums_ref, occz_ref, psums_ref, poccz_ref, w_ref, b_ref,
             out_ref):
    g = pl.program_id(0)
    has_prev = ((g % HB) >= PB).astype(jnp.float32)
    sums = sums_ref[...] - has_prev * psums_ref[...]
    occz = occz_ref[...] - has_prev * poccz_ref[...]
    cnt = jnp.sum(occz, axis=1, keepdims=True)
    mean = sums / jnp.maximum(cnt, 1.0)
    occ = jnp.minimum(occz, 1.0)
    h = lax.dot_general(occ, w_ref[...], (((1,), (1,)), ((), ())),
                        preferred_element_type=jnp.float32) + b_ref[...]
    full = jnp.concatenate([mean, h], axis=1)
    t = full.T
    out_ref[0] = t.reshape(C + EMB, GY, W)


def _prev_map(g):
    return (jnp.where((g % HB) >= PB, g - PB, g), 0)


_tc_finish = pl.pallas_call(
    _tc_body,
    grid=(GRID,),
    in_specs=[
        pl.BlockSpec((GY * W, C), lambda g: (g, 0)),
        pl.BlockSpec((GY * W, ZB), lambda g: (g, 0)),
        pl.BlockSpec((GY * W, C), _prev_map),
        pl.BlockSpec((GY * W, ZB), _prev_map),
        pl.BlockSpec((EMB, ZB), lambda g: (0, 0)),
        pl.BlockSpec((1, EMB), lambda g: (0, 0)),
    ],
    out_specs=pl.BlockSpec((1, C + EMB, GY, W),
                           lambda g: (g // (H // GY), 0, g % (H // GY), 0)),
    out_shape=jax.ShapeDtypeStruct((B, C + EMB, H, W), jnp.float32),
)


def kernel(features, voxel_coords, height_W, height_b):
    vc = voxel_coords
    keys = (vc[:, 0] * HW + vc[:, 2] * W + vc[:, 3]) * ZB + vc[:, 1]
    keys = keys.astype(jnp.int32)
    keys_pad = jnp.concatenate(
        [keys, jnp.full((MP - M,), SENT, jnp.int32)])
    zf = jnp.zeros((CH, C), jnp.float32)
    sums, occz = _sc_scatter(keys_pad, features, zf)
    occz = occz.reshape(NSEG, ZB)
    return _tc_finish(sums, occz, sums, occz, height_W,
                      height_b.reshape(1, EMB))


# speedup vs baseline: 1.4038x; 1.4038x over previous
import jax
import jax.numpy as jnp
from jax import lax
from jax.experimental import pallas as pl
from jax.experimental.pallas import tpu as pltpu
from jax.experimental.pallas import tpu_sc as plsc

H = 256
W = 256
ZB = 16
B = 2
C = 64
EMB = 32
M = 500000
HW = H * W
NSEG = B * HW
RANGE = 8192
NPASS = (NSEG // 2) // RANGE
DUMF = 512
NR = RANGE + DUMF
NZT = NR * ZB
MP = 524288
SHARE = MP // 16
CH = 512
NCH = SHARE // CH
TAIL = M - (M // CH) * CH
FSPAN = RANGE // 16
ZSPAN = RANGE * ZB // 16
SENT = 0x7FFFFFFF


def _sc_body(keys_hbm, feat_hbm, zf_hbm, sums_hbm, occz_hbm,
             kbuf, fbuf, loccb, rkeyb, fbix, zbix, dumf, dumz, onesb,
             dstg, accF, accZ, semk, semf, sems):
    c = lax.axis_index("c")
    s = lax.axis_index("s")
    share_base = s * SHARE
    lane = lax.iota(jnp.int32, 16)

    fb0 = fbuf.at[0]
    pltpu.sync_copy(zf_hbm, fb0)
    zero16 = jnp.zeros((16,), jnp.float32)

    def zb_body(j, carry):
        dstg[pl.ds(j * 16, 16)] = zero16
        return carry

    lax.fori_loop(0, ZSPAN // 16, zb_body, 0)
    for j in range(CH // 16):
        onesb[pl.ds(j * 16, 16)] = jnp.full((16,), 1.0, jnp.float32)
    for q in range(FSPAN // 128):
        row = fbix.at[q]
        for j in range(8):
            row[pl.ds(j * 16, 16)] = lane + (s * FSPAN + q * 128 + j * 16)

    def zbix_body(q, carry):
        row = zbix.at[q]
        for j in range(8):
            row[pl.ds(j * 16, 16)] = lane + (s * ZSPAN + q * 128 + j * 16)
        return carry

    lax.fori_loop(0, ZSPAN // 128, zbix_body, 0)
    for q in range(DUMF // 128):
        dfrow = dumf.at[q]
        for j in range(8):
            e = lane + (q * 128 + j * 16)
            dfrow[pl.ds(j * 16, 16)] = RANGE + (e & (DUMF - 1))
    for q in range(4):
        row = dumz.at[q]
        for j in range(8):
            row[pl.ds(j * 16, 16)] = lane + (RANGE * ZB + q * 128 + j * 16)

    cps = []
    for q in range(FSPAN // 128):
        cps.append(pltpu.async_copy(fb0.at[pl.ds(q * 128, 128), :],
                                    accF.at[fbix.at[q]], sems))
    for q in range(DUMF // 128):
        cps.append(pltpu.async_copy(fb0.at[pl.ds(q * 128, 128), :],
                                    accF.at[dumf.at[q]], sems))
    for q in range(ZSPAN // 128):
        cps.append(pltpu.async_copy(dstg.at[pl.ds(q * 128, 128)],
                                    accZ.at[zbix.at[q]], sems))
    for q in range(4):
        cps.append(pltpu.async_copy(dstg.at[pl.ds(q * 128, 128)],
                                    accZ.at[dumz.at[q]], sems))
    for cp in cps:
        cp.wait()
    plsc.subcore_barrier()

    def issue(n):
        kbase = kbase_of(n)

        @pl.when(kbase < M)
        def _():
            pltpu.async_copy(keys_hbm.at[pl.ds(kbase, CH)],
                             kbuf.at[n % 2], semk)
            fb = fbuf.at[n % 2]
            full = (kbase + CH) <= M

            @pl.when(full)
            def _():
                pltpu.async_copy(feat_hbm.at[pl.ds(kbase, CH), :], fb,
                                 semf)

            @pl.when(jnp.logical_not(full))
            def _():
                pltpu.async_copy(feat_hbm.at[pl.ds(kbase, TAIL), :],
                                 fb.at[pl.ds(0, TAIL), :], semf)
                pltpu.async_copy(feat_hbm.at[pl.ds(0, CH - TAIL), :],
                                 fb.at[pl.ds(TAIL, CH - TAIL), :], semf)

    def kbase_of(n):
        return share_base + n * CH

    def do_pass(p, carry):
        rbase = c * (NSEG // 2) + p * RANGE
        kb16 = rbase * ZB
        issue(jnp.int32(0))

        def chunk_body(i, carry2):
            kbase = kbase_of(i)

            @pl.when(i + 1 < NCH)
            def _():
                issue(i + 1)

            @pl.when(kbase < M)
            def _():
                kb = kbuf.at[i % 2]
                fb = fbuf.at[i % 2]
                pltpu.make_async_copy(keys_hbm.at[pl.ds(0, CH)], kb,
                                      semk).wait()
                for q in range(CH // 128):
                    lrow = loccb.at[q]
                    rrow = rkeyb.at[q]
                    for j in range(8):
                        off = q * 128 + j * 16
                        k = kb[pl.ds(off, 16)]
                        code = lax.shift_right_logical(k, 4)
                        m = jnp.logical_and(code >= rbase,
                                            code < rbase + RANGE)
                        l = lane + off
                        lrow[pl.ds(j * 16, 16)] = jnp.where(
                            m, code - rbase, RANGE + (l & (DUMF - 1)))
                        rrow[pl.ds(j * 16, 16)] = jnp.where(
                            m, k - kb16, RANGE * ZB + (l & 511))
                pltpu.make_async_copy(feat_hbm.at[pl.ds(0, CH), :], fb,
                                      semf).wait()
                adds = []
                for q in range(CH // 128):
                    adds.append(pltpu.async_copy(
                        fb.at[pl.ds(q * 128, 128), :],
                        accF.at[loccb.at[q]], sems, add=True))
                    adds.append(pltpu.async_copy(
                        onesb.at[pl.ds(q * 128, 128)],
                        accZ.at[rkeyb.at[q]], sems, add=True))
                for cp in adds:
                    cp.wait()

            return carry2

        lax.fori_loop(0, NCH, chunk_body, 0)
        plsc.subcore_barrier()
        gs = []
        for q in range(FSPAN // 128):
            gs.append(pltpu.async_copy(accF.at[fbix.at[q]],
                                       fbuf.at[0].at[pl.ds(q * 128, 128), :],
                                       sems))
        for q in range(ZSPAN // 128):
            gs.append(pltpu.async_copy(accZ.at[zbix.at[q]],
                                       dstg.at[pl.ds(q * 128, 128)], sems))
        for cp in gs:
            cp.wait()
        pltpu.sync_copy(fbuf.at[0],
                        sums_hbm.at[pl.ds(rbase + s * FSPAN, FSPAN), :])
        pltpu.sync_copy(dstg,
                        occz_hbm.at[pl.ds(kb16 + s * ZSPAN, ZSPAN)])
        plsc.subcore_barrier()
        return carry

    lax.fori_loop(0, NPASS, do_pass, 0)


_sc_scatter = pl.kernel(
    _sc_body,
    out_type=(jax.ShapeDtypeStruct((NSEG, C), jnp.float32),
              jax.ShapeDtypeStruct((NSEG * ZB,), jnp.float32)),
    mesh=plsc.VectorSubcoreMesh(core_axis_name="c", subcore_axis_name="s"),
    compiler_params=pltpu.CompilerParams(use_tc_tiling_on_sc=False),
    scratch_types=[
        pltpu.VMEM((2, CH), jnp.int32),
        pltpu.VMEM((2, CH, C), jnp.float32),
        pltpu.VMEM((CH // 128, 128), jnp.int32),
        pltpu.VMEM((CH // 128, 128), jnp.int32),
        pltpu.VMEM((FSPAN // 128, 128), jnp.int32),
        pltpu.VMEM((ZSPAN // 128, 128), jnp.int32),
        pltpu.VMEM((DUMF // 128, 128), jnp.int32),
        pltpu.VMEM((4, 128), jnp.int32),
        pltpu.VMEM((CH,), jnp.float32),
        pltpu.VMEM((ZSPAN,), jnp.float32),
        pltpu.VMEM_SHARED((NR, C), jnp.float32),
        pltpu.VMEM_SHARED((NZT,), jnp.float32),
        pltpu.SemaphoreType.DMA,
        pltpu.SemaphoreType.DMA,
        pltpu.SemaphoreType.DMA,
    ],
)

GY = 8
GRID = B * H // GY
PB = RANGE // (GY * W)
HB = (NSEG // 2) // (GY * W)


def _tc_body(sums_ref, occz_ref, psums_ref, poccz_ref, w_ref, b_ref,
             out_ref):
    g = pl.program_id(0)
    has_prev = ((g % HB) >= PB).astype(jnp.float32)
    sums = sums_ref[...] - has_prev * psums_ref[...]
    occz = occz_ref[...] - has_prev * poccz_ref[...]
    cnt = jnp.sum(occz, axis=1, keepdims=True)
    mean = sums / jnp.maximum(cnt, 1.0)
    occ = jnp.minimum(occz, 1.0)
    h = lax.dot_general(occ, w_ref[...], (((1,), (1,)), ((), ())),
                        preferred_element_type=jnp.float32) + b_ref[...]
    full = jnp.concatenate([mean, h], axis=1)
    t = full.T
    out_ref[0] = t.reshape(C + EMB, GY, W)


def _prev_map(g):
    return (jnp.where((g % HB) >= PB, g - PB, g), 0)


_tc_finish = pl.pallas_call(
    _tc_body,
    grid=(GRID,),
    in_specs=[
        pl.BlockSpec((GY * W, C), lambda g: (g, 0)),
        pl.BlockSpec((GY * W, ZB), lambda g: (g, 0)),
        pl.BlockSpec((GY * W, C), _prev_map),
        pl.BlockSpec((GY * W, ZB), _prev_map),
        pl.BlockSpec((EMB, ZB), lambda g: (0, 0)),
        pl.BlockSpec((1, EMB), lambda g: (0, 0)),
    ],
    out_specs=pl.BlockSpec((1, C + EMB, GY, W),
                           lambda g: (g // (H // GY), 0, g % (H // GY), 0)),
    out_shape=jax.ShapeDtypeStruct((B, C + EMB, H, W), jnp.float32),
)


def kernel(features, voxel_coords, height_W, height_b):
    vc = voxel_coords
    keys = (vc[:, 0] * HW + vc[:, 2] * W + vc[:, 3]) * ZB + vc[:, 1]
    keys = keys.astype(jnp.int32)
    keys_pad = jnp.concatenate(
        [keys, jnp.full((MP - M,), SENT, jnp.int32)])
    zf = jnp.zeros((CH, C), jnp.float32)
    sums, occz = _sc_scatter(keys_pad, features, zf)
    occz = occz.reshape(NSEG, ZB)
    return _tc_finish(sums, occz, sums, occz, height_W,
                      height_b.reshape(1, EMB))
